# Initial kernel scaffold; baseline (speedup 1.0000x reference)
import jax
import jax.numpy as jnp
from jax.experimental import pallas as pl


def kernel(x, t, src, tar, n_mask, edge_index, w_time, b_time, W_ih, W_hh, b_ih, b_hh, W_self, b_self, W_nb, b_nb, W_out, b_out):
    raise NotImplementedError("write your pallas kernel here")


# speedup vs baseline: 2.5263x; 2.5263x over previous
import functools

import jax
import jax.numpy as jnp
from jax import lax
from jax.experimental import pallas as pl
from jax.experimental.pallas import tpu as pltpu
from jax.experimental.pallas import tpu_sc as plsc

N = 10000
B = 32
E = 160000
LATENT = 128

NPAD = 10112
RPT = NPAD // 16
CHUNK = 128
NWORK = 16
EPW = 10240
EPAD = NWORK * EPW
NCHUNK = EPW // CHUNK


NP2 = 10240
TN1 = 2048
NJ = NP2 // TN1


def _k1_body(t_ref, m_ref, x_ref, wc_ref, btc_ref, enc_ref, aux_ref):
    NCH = TN1 // 256
    rows = []
    for r in range(8):
        acc = jnp.zeros((1, LATENT), jnp.float32)
        for ch in range(NCH):
            sl = slice(ch * 256, (ch + 1) * 256)
            tt = t_ref[r:r + 1, sl]
            mm = m_ref[r:r + 1, sl]
            c = jnp.cos(wc_ref[...] * tt + btc_ref[...])
            acc += lax.dot_general(
                mm, c, (((1,), (1,)), ((), ())),
                preferred_element_type=jnp.float32)
        rows.append(acc)
    enc_ref[...] = jnp.concatenate(rows, axis=0)
    sx = jnp.sum(m_ref[...] * x_ref[...], axis=1, keepdims=True)
    cnt = jnp.sum(m_ref[...], axis=1, keepdims=True)
    ii = lax.broadcasted_iota(jnp.int32, (8, LATENT), 1)
    aux_ref[...] = (jnp.where(ii == 0, sx, 0.0)
                    + jnp.where(ii == 1, cnt, 0.0))


def _k1(t4, m4, x4, w_col, bt_col):
    return pl.pallas_call(
        _k1_body,
        grid=(NJ * B // 8,),
        in_specs=[
            pl.BlockSpec((8, TN1), lambda i: (i, 0)),
            pl.BlockSpec((8, TN1), lambda i: (i, 0)),
            pl.BlockSpec((8, TN1), lambda i: (i, 0)),
            pl.BlockSpec((LATENT, 1), lambda i: (0, 0)),
            pl.BlockSpec((LATENT, 1), lambda i: (0, 0)),
        ],
        out_specs=[
            pl.BlockSpec((8, LATENT), lambda i: (i, 0)),
            pl.BlockSpec((8, LATENT), lambda i: (i, 0)),
        ],
        out_shape=[
            jax.ShapeDtypeStruct((NJ * B, LATENT), jnp.float32),
            jax.ShapeDtypeStruct((NJ * B, LATENT), jnp.float32),
        ],
    )(t4, m4, x4, w_col, bt_col)


def _k2a_body(x_ref, t_ref, m_ref, src_ref, tar_ref, w_ref, bt_ref,
              wih_ref, bih_ref, bhh_ref, wnb_ref, wself_ref, bself_ref,
              base_ref, cnb_ref, cself_ref, const_ref):
    src = src_ref[0, :]
    tar = tar_ref[0, :]
    nidx = lax.broadcasted_iota(jnp.int32, (B, N), 1)
    msrc = (nidx == src[:, None]).astype(jnp.float32)
    mtar = (nidx == tar[:, None]).astype(jnp.float32)
    x_src = jnp.sum(x_ref[...] * msrc, axis=1)
    x_tar = jnp.sum(x_ref[...] * mtar, axis=1)
    t_src = jnp.sum(t_ref[...] * msrc, axis=1)
    t_tar = jnp.sum(t_ref[...] * mtar, axis=1)

    w = w_ref[0, :].reshape(1, -1)
    bt = bt_ref[0, :].reshape(1, -1)
    dt_src = jnp.cos(t_src[:, None] * w + bt)
    dt_tar = jnp.cos(t_tar[:, None] * w + bt)

    wih_x0 = wih_ref[0, :].reshape(1, -1)
    wih_x1 = wih_ref[1, :].reshape(1, -1)
    wih_dt = wih_ref[2 + LATENT:, :]
    bih = bih_ref[0, :].reshape(1, -1)
    bhh = bhh_ref[0, :].reshape(1, -1)

    def gru(xa, xb, dt):
        gi = (xa[:, None] * wih_x0 + xb[:, None] * wih_x1 + bih
              + jnp.dot(dt, wih_dt, preferred_element_type=jnp.float32))
        r = jax.nn.sigmoid(gi[:, :LATENT] + bhh[:, :LATENT])
        z = jax.nn.sigmoid(gi[:, LATENT:2 * LATENT] + bhh[:, LATENT:2 * LATENT])
        n = jnp.tanh(gi[:, 2 * LATENT:] + r * bhh[:, 2 * LATENT:])
        return (1.0 - z) * n

    new_src = gru(x_src, x_tar, dt_src)
    new_tar = gru(x_tar, x_src, dt_tar)
    vals = jnp.concatenate([new_src, new_tar], axis=0)
    idx64 = jnp.concatenate([src, tar], axis=0)

    p = lax.broadcasted_iota(jnp.int32, (2 * B, 2 * B), 0)
    q = lax.broadcasted_iota(jnp.int32, (2 * B, 2 * B), 1)
    later_same = jnp.logical_and(q > p, idx64[None, :] == idx64[:, None])
    is_final = jnp.logical_not(jnp.any(later_same, axis=1))
    valsf = vals * is_final[:, None].astype(jnp.float32)

    wnb_mem = wnb_ref[1:1 + LATENT, :]
    wself_mem = wself_ref[1:1 + LATENT, :]
    wself_enc = wself_ref[1 + LATENT:, :]
    bself = bself_ref[0, :].reshape(1, -1)

    cnb_ref[...] = jnp.dot(valsf, wnb_mem, preferred_element_type=jnp.float32)
    cself = jnp.dot(valsf, wself_mem, preferred_element_type=jnp.float32)
    cself_ref[...] = cself

    cosb = jnp.cos(bt)
    const_row = jnp.dot(cosb, wself_enc,
                        preferred_element_type=jnp.float32) + bself
    const_ref[...] = const_row

    sel = (idx64[None, :] == tar[:, None]).astype(jnp.float32)
    tar_mem = jnp.dot(sel, valsf, preferred_element_type=jnp.float32)
    onehot = (lax.broadcasted_iota(jnp.int32, (N, 2 * B), 0)
              == idx64[None, :]).astype(jnp.float32)
    mg = jnp.dot(m_ref[...], onehot, preferred_element_type=jnp.float32)
    hsum_mem = jnp.dot(mg, valsf, preferred_element_type=jnp.float32)
    wself_x = wself_ref[0, :].reshape(1, -1)
    base = (x_tar[:, None] * wself_x
            + jnp.dot(tar_mem, wself_mem, preferred_element_type=jnp.float32)
            + const_row
            + jnp.dot(hsum_mem, wnb_mem, preferred_element_type=jnp.float32))
    base_ref[...] = base


def _k2a(x2, t2, m2, src2, tar2, w_time, b_time, W_ih, b_ih, b_hh,
         W_nb, W_self, b_self):
    return pl.pallas_call(
        _k2a_body,
        out_shape=[
            jax.ShapeDtypeStruct((B, LATENT), jnp.float32),
            jax.ShapeDtypeStruct((2 * B, LATENT), jnp.float32),
            jax.ShapeDtypeStruct((2 * B, LATENT), jnp.float32),
            jax.ShapeDtypeStruct((1, LATENT), jnp.float32),
        ],
    )(x2, t2, m2, src2, tar2, w_time, b_time, W_ih, b_ih, b_hh,
      W_nb, W_self, b_self)


def _k2b_body(base_ref, enc_ref, aux_ref, wnb_ref, bnb_ref, wout_ref,
              bout_ref, out_ref):
    enc = enc_ref[0:B, :]
    aux = aux_ref[0:B, :]
    for j in range(1, NJ):
        enc = enc + enc_ref[j * B:(j + 1) * B, :]
        aux = aux + aux_ref[j * B:(j + 1) * B, :]
    sx = aux[:, 0:1]
    cnt = aux[:, 1:2]
    wnb_x = wnb_ref[0, :].reshape(1, -1)
    wnb_enc = wnb_ref[1 + LATENT:, :]
    agg = (sx * wnb_x
           + jnp.dot(enc, wnb_enc, preferred_element_type=jnp.float32)
           + cnt * bnb_ref[0, :].reshape(1, -1))
    z = jax.nn.relu(base_ref[...] + agg)
    out_ref[...] = jnp.dot(z, wout_ref[...],
                           preferred_element_type=jnp.float32) + bout_ref[0, 0]


def _k2b(base, enc, aux, W_nb, b_nb, W_out, b_out2):
    return pl.pallas_call(
        _k2b_body,
        out_shape=jax.ShapeDtypeStruct((B, 1), jnp.float32),
    )(base, enc, aux, W_nb, b_nb, W_out, b_out2)


def _k3_body(xl_ref, tl_ref, w_ref, bt_ref, wnb_ref, bnb_ref, cnb_ref,
             idx_ref, out_ref):
    i = pl.program_id(0)
    Tn = out_ref.shape[0]
    enc = jnp.cos(tl_ref[...] * w_ref[0, :].reshape(1, -1)
                  + bt_ref[0, :].reshape(1, -1))
    wnb_enc = wnb_ref[1 + LATENT:, :]
    nb = (jnp.dot(enc, wnb_enc, preferred_element_type=jnp.float32)
          + xl_ref[...] * wnb_ref[0, :].reshape(1, -1)
          + bnb_ref[0, :].reshape(1, -1))
    rows = i * Tn + lax.broadcasted_iota(jnp.int32, (Tn, 2 * B), 0)
    M = (rows == idx_ref[0, :][None, :]).astype(jnp.float32)
    out_ref[...] = nb + jnp.dot(M, cnb_ref[...],
                                preferred_element_type=jnp.float32)


def _k3(xl, tl, w_time, b_time, W_nb, b_nb, contrib_nb, idxrep):
    Tn = 1000
    return pl.pallas_call(
        _k3_body,
        grid=(N // Tn,),
        in_specs=[
            pl.BlockSpec((Tn, 1), lambda i: (i, 0)),
            pl.BlockSpec((Tn, 1), lambda i: (i, 0)),
            pl.BlockSpec((1, LATENT), lambda i: (0, 0)),
            pl.BlockSpec((1, LATENT), lambda i: (0, 0)),
            pl.BlockSpec((257, LATENT), lambda i: (0, 0)),
            pl.BlockSpec((1, LATENT), lambda i: (0, 0)),
            pl.BlockSpec((2 * B, LATENT), lambda i: (0, 0)),
            pl.BlockSpec((8, 2 * B), lambda i: (0, 0)),
        ],
        out_specs=pl.BlockSpec((Tn, LATENT), lambda i: (i, 0)),
        out_shape=jax.ShapeDtypeStruct((N, LATENT), jnp.float32),
    )(xl, tl, w_time, b_time, W_nb, b_nb, contrib_nb, idxrep)


def _k4_body(nb_hbm, jidx_hbm, iidx_hbm, zeros_hbm, out_hbm,
             jv, iv, rows, acc, sem):
    s = lax.axis_index("s")

    pltpu.sync_copy(zeros_hbm.at[pl.ds(s * RPT, RPT)],
                    acc.at[pl.ds(s * RPT, RPT)])
    plsc.subcore_barrier()

    def body(ch, carry):
        base = s * EPW + ch * CHUNK
        pltpu.sync_copy(jidx_hbm.at[pl.ds(base, CHUNK)], jv)
        pltpu.sync_copy(iidx_hbm.at[pl.ds(base, CHUNK)], iv)
        pltpu.async_copy(nb_hbm.at[jv], rows, sem).wait()
        pltpu.sync_copy(rows, acc.at[iv], add=True)
        return carry

    lax.fori_loop(0, NCHUNK, body, 0)
    plsc.subcore_barrier()

    pltpu.sync_copy(acc.at[pl.ds(s * RPT, RPT)],
                    out_hbm.at[pl.ds(s * RPT, RPT)])


def _k4(nb, jidx, iidx, zeros):
    mesh = plsc.VectorSubcoreMesh(core_axis_name="c", subcore_axis_name="s",
                                  num_cores=1)
    k = functools.partial(
        pl.kernel,
        out_type=jax.ShapeDtypeStruct((NPAD, LATENT), jnp.float32),
        mesh=mesh,
        scratch_types=[
            pltpu.VMEM((CHUNK,), jnp.int32),
            pltpu.VMEM((CHUNK,), jnp.int32),
            pltpu.VMEM((CHUNK, LATENT), jnp.float32),
            pltpu.VMEM_SHARED((NPAD, LATENT), jnp.float32),
            pltpu.SemaphoreType.DMA,
        ],
    )(_k4_body)
    return k(nb, jidx, iidx, zeros)


def _k5_body(p0_ref, xl_ref, wself_ref, const_ref, cself_ref,
             idx_ref, wout_ref, bout_ref, out_ref):
    i = pl.program_id(0)
    Tn = out_ref.shape[0]
    rows = i * Tn + lax.broadcasted_iota(jnp.int32, (Tn, 2 * B), 0)
    M = (rows == idx_ref[0, :][None, :]).astype(jnp.float32)
    z = jax.nn.relu(p0_ref[...]
                    + xl_ref[...] * wself_ref[0, :].reshape(1, -1)
                    + const_ref[...]
                    + jnp.dot(M, cself_ref[...],
                              preferred_element_type=jnp.float32))
    out_ref[...] = jnp.dot(z, wout_ref[...],
                           preferred_element_type=jnp.float32) + bout_ref[0, 0]


def _k5(p0, xl, W_self, const_row, contrib_self, idxrep, W_out, b_out2):
    Tn = 1000
    return pl.pallas_call(
        _k5_body,
        grid=(N // Tn,),
        in_specs=[
            pl.BlockSpec((Tn, LATENT), lambda i: (i, 0)),
            pl.BlockSpec((Tn, 1), lambda i: (i, 0)),
            pl.BlockSpec((257, LATENT), lambda i: (0, 0)),
            pl.BlockSpec((1, LATENT), lambda i: (0, 0)),
            pl.BlockSpec((2 * B, LATENT), lambda i: (0, 0)),
            pl.BlockSpec((8, 2 * B), lambda i: (0, 0)),
            pl.BlockSpec((LATENT, 1), lambda i: (0, 0)),
            pl.BlockSpec((1, 1), lambda i: (0, 0)),
        ],
        out_specs=pl.BlockSpec((Tn, 1), lambda i: (i, 0)),
        out_shape=jax.ShapeDtypeStruct((N, 1), jnp.float32),
    )(p0, xl, W_self, const_row, contrib_self, idxrep, W_out, b_out2)


def kernel(x, t, src, tar, n_mask, edge_index, w_time, b_time, W_ih, W_hh,
           b_ih, b_hh, W_self, b_self, W_nb, b_nb, W_out, b_out):
    x2 = x[:, :, 0]
    t2 = t[:, :, 0]
    m2 = n_mask.astype(jnp.float32)
    src2 = src.reshape(1, B).astype(jnp.int32)
    tar2 = tar.reshape(1, B).astype(jnp.int32)
    bt2 = b_time.reshape(1, LATENT)
    bih2 = b_ih.reshape(1, -1)
    bhh2 = b_hh.reshape(1, -1)
    bself2 = b_self.reshape(1, LATENT)
    bnb2 = b_nb.reshape(1, LATENT)
    bout2 = b_out.reshape(1, 1)

    idx64 = jnp.concatenate([src2[0], tar2[0]], axis=0)
    idxrep = jnp.broadcast_to(idx64[None, :], (8, 2 * B))

    padn = NP2 - N

    def _tile4(a):
        return (jnp.pad(a, ((0, 0), (0, padn)))
                .reshape(B, NJ, TN1).transpose(1, 0, 2).reshape(NJ * B, TN1))

    w_col = w_time.reshape(LATENT, 1)
    bt_col = b_time.reshape(LATENT, 1)
    enc160, aux160 = _k1(_tile4(t2), _tile4(m2), _tile4(x2), w_col, bt_col)
    base, contrib_nb, contrib_self, const_row = _k2a(
        x2, t2, m2, src2, tar2, w_time, bt2, W_ih, bih2, bhh2,
        W_nb, W_self, bself2)
    step_logit = _k2b(base, enc160, aux160, W_nb, bnb2, W_out, bout2)

    xl = x[0]
    tl = t[-1]
    nb = _k3(xl, tl, w_time, bt2, W_nb, bnb2, contrib_nb, idxrep)

    key = edge_index[0].astype(jnp.int32) * N + edge_index[1].astype(jnp.int32)
    skey = jnp.sort(key)
    uniq = jnp.concatenate(
        [jnp.ones((1,), jnp.bool_), skey[1:] != skey[:-1]])
    i_s = skey // N
    j_s = skey % N
    i_eff = jnp.where(uniq, i_s, N)
    pad = EPAD - E
    jidx = jnp.concatenate([j_s, jnp.zeros((pad,), jnp.int32)])
    iidx = jnp.concatenate([i_eff, jnp.full((pad,), N, jnp.int32)])
    zeros = jnp.zeros((NPAD, LATENT), jnp.float32)

    partials = _k4(nb, jidx, iidx, zeros)
    p0 = partials[:N]

    last_logit = _k5(p0, xl, W_self, const_row, contrib_self, idxrep,
                     W_out, bout2)
    return (step_logit, last_logit)
